# baseline (speedup 1.0000x reference)
import functools

import jax
import jax.numpy as jnp
from jax import lax
from jax.experimental import pallas as pl
from jax.experimental.pallas import tpu as pltpu
from jax.experimental.pallas import tpu_sc as plsc

NC = 2
NS = 16
NW = NC * NS

NTOKEN = 1000000
D = 64
B = 16384 * 20
BPW = B // NW
IDX_MINOR = 128
K = 4
ROWS_PER_OUTER = K * IDX_MINOR
NOUTER = BPW // ROWS_PER_OUTER
NSTREAMS = BPW // IDX_MINOR

@functools.cache
def _build_embedding_gather():
    mesh = plsc.VectorSubcoreMesh(core_axis_name="c", subcore_axis_name="s")

    @functools.partial(
        pl.kernel,
        mesh=mesh,
        compiler_params=pltpu.CompilerParams(use_tc_tiling_on_sc=False),
        out_type=jax.ShapeDtypeStruct((B, D), jnp.float32),
        scratch_types=[
            pltpu.VMEM((NSTREAMS, IDX_MINOR), jnp.int32),
            pltpu.VMEM((ROWS_PER_OUTER, D), jnp.float32),
            pltpu.SemaphoreType.DMA,
        ],
    )
    def _embedding_gather(idx_hbm, table_hbm, out_hbm, idx_v, rows_v, gsem):
        wid = lax.axis_index("s") * NC + lax.axis_index("c")
        pltpu.sync_copy(idx_hbm.at[wid], idx_v)

        def outer(g, carry):
            copies = []
            for j in range(K):
                cp = pltpu.async_copy(
                    table_hbm.at[idx_v.at[g * K + j]],
                    rows_v.at[pl.ds(j * IDX_MINOR, IDX_MINOR)],
                    gsem,
                )
                copies.append(cp)
            for cp in copies:
                cp.wait()
            base = wid * BPW + g * ROWS_PER_OUTER
            pltpu.sync_copy(rows_v, out_hbm.at[pl.ds(base, ROWS_PER_OUTER)])
            return carry

        lax.fori_loop(0, NOUTER, outer, 0)

    return _embedding_gather


def kernel(input, encoder_weight):
    idx = jnp.reshape(input.astype(jnp.int32), (NW, NSTREAMS, IDX_MINOR))
    out = _build_embedding_gather()(idx, encoder_weight)
    return jnp.reshape(out, (*input.shape, D))
